# baseline (speedup 1.0000x reference)
import functools

import jax
import jax.numpy as jnp
from jax import lax
from jax.experimental import pallas as pl
from jax.experimental.pallas import tpu as pltpu
from jax.experimental.pallas import tpu_sc as plsc

B = 128
V = 100000
NC, NS, L = 2, 16, 16
NW = NC * NS
RPW = B // NW
NVEC = V // L
NB = 256
HSZ = NB * L
CPL = 768
CAP = CPL * L
UNROLL = 10


def _fkey(x):
    b = plsc.bitcast(x, jnp.int32)
    return jnp.where(b < 0, (b & 0x7FFFFFFF) ^ (-1), b)


def _unkey(s):
    b = jnp.where(s < 0, (s ^ (-1)) | jnp.int32(-(2 ** 31)), s)
    return plsc.bitcast(b, jnp.float32)


def _uloop(body, init):
    if isinstance(init, int):
        init = jnp.int32(init)
    return plsc.parallel_loop(0, NVEC, 1, unroll=UNROLL, carry=init)(body)


def _zero(ref, zval):
    def bd(i, c):
        ref[pl.ds(i * L, L)] = zval
        return c
    plsc.parallel_loop(0, HSZ // L, 1, unroll=8, carry=jnp.int32(0))(bd)


def _select(ref, target, zscal):
    zv = jnp.full((L,), zscal)
    onev = jnp.ones((L,), jnp.int32)
    zci = jnp.zeros((L,), jnp.int32)

    def bd(jj, c):
        carry, cntv, abv = c
        j = 15 - jj
        tot = zv
        for l in range(L):
            tot = tot + ref[pl.ds(l * NB + j * L, L)]
        csum = lax.rev(plsc.cumsum(lax.rev(tot, (0,))), (0,))
        cum = csum + carry
        mask = cum >= target
        cntv = cntv + jnp.where(mask, onev, zci)
        abv = jnp.maximum(abv, jnp.where(mask, zv, cum))
        return (jnp.max(cum), cntv, abv)

    total, cntv, abv = lax.fori_loop(0, 16, bd, (zscal, zci, zv))
    bkt = jnp.maximum(jnp.sum(cntv) - 1, 0)
    return bkt, jnp.max(abv)


def _body(lg, scal, out, xbuf, hist_i, hist_f, cand, ssp):
    cid = lax.axis_index("c")
    sid = lax.axis_index("s")
    wid = sid * NC + cid
    lanes = lax.iota(jnp.int32, L)
    lanebase = lanes * NB
    lanebase128 = lanebase + 128
    lanesC = lanes * CPL
    ones_i = jnp.ones((L,), jnp.int32)
    zeros_i = jnp.zeros((L,), jnp.int32)
    zeros_f = jnp.zeros((L,), jnp.float32)
    neg_inf = jnp.full((L,), -jnp.inf, jnp.float32)

    def row_body(r, c):
        row = wid * RPW + r
        pltpu.sync_copy(lg.at[row], xbuf)
        pltpu.sync_copy(scal.at[row], ssp)
        tval = ssp[pl.ds(0, L)]
        kt = jnp.max(plsc.bitcast(ssp[pl.ds(L, L)], jnp.int32))
        ps = jnp.max(ssp[pl.ds(2 * L, L)])

        _zero(hist_i, zeros_i)

        def p1(i, mx):
            v = xbuf[pl.ds(i * L, L)]
            s = _fkey(v / tval)
            xbuf[pl.ds(i * L, L)] = plsc.bitcast(s, jnp.float32)
            plsc.addupdate_scatter(hist_i, [lanebase128 + (s >> 24)], ones_i)
            return jnp.maximum(mx, s)

        mxk = _uloop(p1, jnp.full((L,), jnp.int32(-(2 ** 31))))
        smax = jnp.max(mxk)
        mxs = _unkey(jnp.full((L,), smax))

        def key_at(i):
            return plsc.bitcast(xbuf[pl.ds(i * L, L)], jnp.int32)

        bkt, above = _select(hist_i, kt, jnp.int32(0))
        pref = bkt - 128
        tgt = kt - above
        _zero(hist_i, zeros_i)

        def pc1(i, c2, pref=pref):
            s = key_at(i)
            m = (s >> 24) == pref
            plsc.addupdate_scatter(
                hist_i, [lanebase + ((s >> 16) & 0xFF)], ones_i, mask=m)
            return c2

        _uloop(pc1, 0)
        bkt, above = _select(hist_i, tgt, jnp.int32(0))
        pref16 = (pref << 8) | bkt
        tgt = tgt - above

        def ccomp(i, off, pref16=pref16):
            s = key_at(i)
            m = (s >> 16) == pref16
            addr = lanesC + jnp.minimum(off, CPL - 1)
            plsc.store_scatter(cand, [addr], i * L + lanes, mask=m)
            return off + jnp.where(m, ones_i, zeros_i)

        offv = _uloop(ccomp, zeros_i)
        ntrip = jnp.max(offv)

        def gather_keys(j, offs):
            valid = j < offs
            idxv = plsc.load_gather(cand, [lanesC + j], mask=valid)
            sv = plsc.bitcast(
                plsc.load_gather(xbuf, [idxv], mask=valid), jnp.int32)
            return valid, sv

        def fast_c():
            _zero(hist_i, zeros_i)

            def l2(j, c2):
                valid, sv = gather_keys(j, offv)
                plsc.addupdate_scatter(
                    hist_i, [lanebase + ((sv >> 8) & 0xFF)], ones_i,
                    mask=valid)
                return c2

            lax.fori_loop(0, ntrip, l2, 0)
            b2, ab2 = _select(hist_i, tgt, jnp.int32(0))
            t3 = tgt - ab2
            _zero(hist_i, zeros_i)

            def l3(j, c2):
                valid, sv = gather_keys(j, offv)
                m = valid & (((sv >> 8) & 0xFF) == b2)
                plsc.addupdate_scatter(
                    hist_i, [lanebase + (sv & 0xFF)], ones_i, mask=m)
                return c2

            lax.fori_loop(0, ntrip, l3, 0)
            b3, _ = _select(hist_i, t3, jnp.int32(0))
            return (b2 << 8) | b3

        def slow_c():
            pref_l, tg = pref16, tgt
            for lev in (2, 3):
                sp, sb = 32 - 8 * lev, 24 - 8 * lev
                _zero(hist_i, zeros_i)

                def pc(i, c2, sp=sp, sb=sb, pref_l=pref_l):
                    s = key_at(i)
                    m = (s >> sp) == pref_l
                    plsc.addupdate_scatter(
                        hist_i, [lanebase + ((s >> sb) & 0xFF)], ones_i,
                        mask=m)
                    return c2

                _uloop(pc, 0)
                bq, abq = _select(hist_i, tg, jnp.int32(0))
                pref_l = (pref_l << 8) | bq
                tg = tg - abq
            return pref_l & 0xFFFF

        low16 = lax.cond(ntrip <= CPL, fast_c, slow_c)
        s_thr = (pref16 << 16) | low16

        _zero(hist_f, zeros_f)

        def pm0(i, zacc):
            s = key_at(i)
            m = s >= s_thr
            e = jnp.exp(_unkey(s) - mxs)
            plsc.addupdate_scatter(hist_f, [lanebase128 + (s >> 24)],
                                   e, mask=m)
            return zacc + jnp.where(m, e, zeros_f)

        zv = _uloop(pm0, zeros_f)
        ptarget = ps * jnp.sum(zv)
        bktm, fabove = _select(hist_f, ptarget, jnp.float32(0))
        prefm = bktm - 128
        ftgt = ptarget - fabove
        _zero(hist_f, zeros_f)

        def pm1(i, c2, prefm=prefm):
            s = key_at(i)
            m = ((s >> 24) == prefm) & (s >= s_thr)
            e = jnp.exp(_unkey(s) - mxs)
            plsc.addupdate_scatter(
                hist_f, [lanebase + ((s >> 16) & 0xFF)], e, mask=m)
            return c2

        _uloop(pm1, 0)
        bktm, fabove = _select(hist_f, ftgt, jnp.float32(0))
        prefm16 = (prefm << 8) | bktm
        ftgt = ftgt - fabove

        def mcomp(i, off, prefm16=prefm16):
            s = key_at(i)
            m = ((s >> 16) == prefm16) & (s >= s_thr)
            addr = lanesC + jnp.minimum(off, CPL - 1)
            plsc.store_scatter(cand, [addr], i * L + lanes, mask=m)
            return off + jnp.where(m, ones_i, zeros_i)

        offm = _uloop(mcomp, zeros_i)
        mtrip = jnp.max(offm)

        def fast_m():
            _zero(hist_f, zeros_f)

            def m2(j, c2):
                valid, sv = gather_keys(j, offm)
                e = jnp.exp(_unkey(sv) - mxs)
                plsc.addupdate_scatter(
                    hist_f, [lanebase + ((sv >> 8) & 0xFF)], e, mask=valid)
                return c2

            lax.fori_loop(0, mtrip, m2, 0)
            b2, ab2 = _select(hist_f, ftgt, jnp.float32(0))
            t3 = ftgt - ab2
            _zero(hist_f, zeros_f)

            def m3(j, c2):
                valid, sv = gather_keys(j, offm)
                m = valid & (((sv >> 8) & 0xFF) == b2)
                e = jnp.exp(_unkey(sv) - mxs)
                plsc.addupdate_scatter(
                    hist_f, [lanebase + (sv & 0xFF)], e, mask=m)
                return c2

            lax.fori_loop(0, mtrip, m3, 0)
            b3, _ = _select(hist_f, t3, jnp.float32(0))
            return (b2 << 8) | b3

        def slow_m():
            pref_l, tg = prefm16, ftgt
            for lev in (2, 3):
                sp, sb = 32 - 8 * lev, 24 - 8 * lev
                _zero(hist_f, zeros_f)

                def pm(i, c2, sp=sp, sb=sb, pref_l=pref_l):
                    s = key_at(i)
                    m = ((s >> sp) == pref_l) & (s >= s_thr)
                    e = jnp.exp(_unkey(s) - mxs)
                    plsc.addupdate_scatter(
                        hist_f, [lanebase + ((s >> sb) & 0xFF)], e, mask=m)
                    return c2

                _uloop(pm, 0)
                bq, abq = _select(hist_f, tg, jnp.float32(0))
                pref_l = (pref_l << 8) | bq
                tg = tg - abq
            return pref_l & 0xFFFF

        low16m = lax.cond(mtrip <= CPL, fast_m, slow_m)
        s_tp = (prefm16 << 16) | low16m

        smaxv = jnp.full((L,), smax)

        def po(i, c2):
            s = key_at(i)
            keep = (s >= s_tp) | (s == smaxv)
            xbuf[pl.ds(i * L, L)] = jnp.where(keep, _unkey(s), neg_inf)
            return c2

        _uloop(po, 0)
        pltpu.sync_copy(xbuf, out.at[row])
        return c

    lax.fori_loop(0, RPW, row_body, 0)


_call = functools.partial(
    pl.kernel,
    out_type=jax.ShapeDtypeStruct((B, V), jnp.float32),
    mesh=plsc.VectorSubcoreMesh(core_axis_name="c", subcore_axis_name="s",
                                num_cores=NC, num_subcores=NS),
    scratch_types=[
        pltpu.VMEM((V,), jnp.float32),
        pltpu.VMEM((HSZ,), jnp.int32),
        pltpu.VMEM((HSZ,), jnp.float32),
        pltpu.VMEM((CAP,), jnp.int32),
        pltpu.VMEM((3 * L,), jnp.float32),
    ],
    compiler_params=pltpu.CompilerParams(needs_layout_passes=False),
)(_body)


def kernel(logits, temperatures, k, p):
    t = temperatures.astype(jnp.float32)
    kf = jax.lax.bitcast_convert_type(k.astype(jnp.int32), jnp.float32)
    pf = p.astype(jnp.float32)
    scal = jnp.stack([t, kf, pf], axis=1)
    scal = jnp.broadcast_to(scal[:, :, None], (B, 3, L)).reshape(B, 3 * L)
    return _call(logits.astype(jnp.float32), scal)


# speedup vs baseline: 1.0690x; 1.0690x over previous
import functools

import jax
import jax.numpy as jnp
from jax import lax
from jax.experimental import pallas as pl
from jax.experimental.pallas import tpu as pltpu
from jax.experimental.pallas import tpu_sc as plsc

B = 128
V = 100000
NC, NS, L = 2, 16, 16
NW = NC * NS
RPW = B // NW
NVEC = V // L
NB = 256
HSZ = NB * L
CPL = 768
CAP = CPL * L
UNROLL = 10


def _fkey(x):
    b = plsc.bitcast(x, jnp.int32)
    return jnp.where(b < 0, (b & 0x7FFFFFFF) ^ (-1), b)


def _unkey(s):
    b = jnp.where(s < 0, (s ^ (-1)) | jnp.int32(-(2 ** 31)), s)
    return plsc.bitcast(b, jnp.float32)


def _uloop(body, init):
    if isinstance(init, int):
        init = jnp.int32(init)
    return plsc.parallel_loop(0, NVEC, 1, unroll=UNROLL, carry=init)(body)


def _zero(ref, zval):
    def bd(i, c):
        ref[pl.ds(i * L, L)] = zval
        return c
    plsc.parallel_loop(0, HSZ // L, 1, unroll=8, carry=jnp.int32(0))(bd)


def _select(ref, target, zscal):
    zv = jnp.full((L,), zscal)
    onev = jnp.ones((L,), jnp.int32)
    zci = jnp.zeros((L,), jnp.int32)

    def bd(jj, c):
        carry, cntv, abv = c
        j = 15 - jj
        tot = zv
        for l in range(L):
            tot = tot + ref[pl.ds(l * NB + j * L, L)]
        csum = lax.rev(plsc.cumsum(lax.rev(tot, (0,))), (0,))
        cum = csum + carry
        mask = cum >= target
        cntv = cntv + jnp.where(mask, onev, zci)
        abv = jnp.maximum(abv, jnp.where(mask, zv, cum))
        return (jnp.max(cum), cntv, abv)

    total, cntv, abv = lax.fori_loop(0, 16, bd, (zscal, zci, zv))
    bkt = jnp.maximum(jnp.sum(cntv) - 1, 0)
    return bkt, jnp.max(abv)


def _body(lg, scal, out, xbuf, hist_i, hist_f, cand, ssp):
    cid = lax.axis_index("c")
    sid = lax.axis_index("s")
    wid = sid * NC + cid
    lanes = lax.iota(jnp.int32, L)
    lanebase = lanes * NB
    lanebase128 = lanebase + 128
    lanesC = lanes * CPL
    ones_i = jnp.ones((L,), jnp.int32)
    zeros_i = jnp.zeros((L,), jnp.int32)
    zeros_f = jnp.zeros((L,), jnp.float32)
    neg_inf = jnp.full((L,), -jnp.inf, jnp.float32)

    def row_body(r, c):
        row = wid * RPW + r
        pltpu.sync_copy(lg.at[row], xbuf)
        pltpu.sync_copy(scal.at[row], ssp)
        tval = ssp[pl.ds(0, L)]
        kt = jnp.max(plsc.bitcast(ssp[pl.ds(L, L)], jnp.int32))
        ps = jnp.max(ssp[pl.ds(2 * L, L)])

        _zero(hist_i, zeros_i)

        def p1(i, mx):
            v = xbuf[pl.ds(i * L, L)]
            x = v / tval
            xbuf[pl.ds(i * L, L)] = x
            s = _fkey(x)
            plsc.addupdate_scatter(hist_i, [lanebase128 + (s >> 24)], ones_i)
            return jnp.maximum(mx, x)

        mxv = _uloop(p1, neg_inf)
        mxs = jnp.max(mxv)
        smax = jnp.max(_fkey(jnp.full((L,), mxs)))

        def key_at(i):
            return _fkey(xbuf[pl.ds(i * L, L)])

        bkt, above = _select(hist_i, kt, jnp.int32(0))
        pref = bkt - 128
        tgt = kt - above
        _zero(hist_i, zeros_i)

        def pc1(i, c2, pref=pref):
            s = key_at(i)
            m = (s >> 24) == pref
            plsc.addupdate_scatter(
                hist_i, [lanebase + ((s >> 16) & 0xFF)], ones_i, mask=m)
            return c2

        _uloop(pc1, 0)
        bkt, above = _select(hist_i, tgt, jnp.int32(0))
        pref16 = (pref << 8) | bkt
        tgt = tgt - above

        def ccomp(i, off, pref16=pref16):
            s = key_at(i)
            m = (s >> 16) == pref16
            addr = lanesC + jnp.minimum(off, CPL - 1)
            plsc.store_scatter(cand, [addr], i * L + lanes, mask=m)
            return off + jnp.where(m, ones_i, zeros_i)

        offv = _uloop(ccomp, zeros_i)
        ntrip = jnp.max(offv)

        def gather_keys(j, offs):
            valid = j < offs
            idxv = plsc.load_gather(cand, [lanesC + j], mask=valid)
            xv = plsc.load_gather(xbuf, [idxv], mask=valid)
            return valid, xv, _fkey(xv)

        def fast_c():
            _zero(hist_i, zeros_i)

            def l2(j, c2):
                valid, _, sv = gather_keys(j, offv)
                plsc.addupdate_scatter(
                    hist_i, [lanebase + ((sv >> 8) & 0xFF)], ones_i,
                    mask=valid)
                return c2

            lax.fori_loop(0, ntrip, l2, 0)
            b2, ab2 = _select(hist_i, tgt, jnp.int32(0))
            t3 = tgt - ab2
            _zero(hist_i, zeros_i)

            def l3(j, c2):
                valid, _, sv = gather_keys(j, offv)
                m = valid & (((sv >> 8) & 0xFF) == b2)
                plsc.addupdate_scatter(
                    hist_i, [lanebase + (sv & 0xFF)], ones_i, mask=m)
                return c2

            lax.fori_loop(0, ntrip, l3, 0)
            b3, _ = _select(hist_i, t3, jnp.int32(0))
            return (b2 << 8) | b3

        def slow_c():
            pref_l, tg = pref16, tgt
            for lev in (2, 3):
                sp, sb = 32 - 8 * lev, 24 - 8 * lev
                _zero(hist_i, zeros_i)

                def pc(i, c2, sp=sp, sb=sb, pref_l=pref_l):
                    s = key_at(i)
                    m = (s >> sp) == pref_l
                    plsc.addupdate_scatter(
                        hist_i, [lanebase + ((s >> sb) & 0xFF)], ones_i,
                        mask=m)
                    return c2

                _uloop(pc, 0)
                bq, abq = _select(hist_i, tg, jnp.int32(0))
                pref_l = (pref_l << 8) | bq
                tg = tg - abq
            return pref_l & 0xFFFF

        low16 = lax.cond(ntrip <= CPL, fast_c, slow_c)
        s_thr = (pref16 << 16) | low16

        _zero(hist_f, zeros_f)

        def pm0(i, zacc):
            x = xbuf[pl.ds(i * L, L)]
            s = _fkey(x)
            m = s >= s_thr
            e = jnp.exp(x - mxs)
            plsc.addupdate_scatter(hist_f, [lanebase128 + (s >> 24)],
                                   e, mask=m)
            return zacc + jnp.where(m, e, zeros_f)

        zv = _uloop(pm0, zeros_f)
        ptarget = ps * jnp.sum(zv)
        bktm, fabove = _select(hist_f, ptarget, jnp.float32(0))
        prefm = bktm - 128
        ftgt = ptarget - fabove
        _zero(hist_f, zeros_f)

        def pm1(i, c2, prefm=prefm):
            x = xbuf[pl.ds(i * L, L)]
            s = _fkey(x)
            m = ((s >> 24) == prefm) & (s >= s_thr)
            e = jnp.exp(x - mxs)
            plsc.addupdate_scatter(
                hist_f, [lanebase + ((s >> 16) & 0xFF)], e, mask=m)
            return c2

        _uloop(pm1, 0)
        bktm, fabove = _select(hist_f, ftgt, jnp.float32(0))
        prefm16 = (prefm << 8) | bktm
        ftgt = ftgt - fabove

        def mcomp(i, off, prefm16=prefm16):
            s = key_at(i)
            m = ((s >> 16) == prefm16) & (s >= s_thr)
            addr = lanesC + jnp.minimum(off, CPL - 1)
            plsc.store_scatter(cand, [addr], i * L + lanes, mask=m)
            return off + jnp.where(m, ones_i, zeros_i)

        offm = _uloop(mcomp, zeros_i)
        mtrip = jnp.max(offm)

        def fast_m():
            _zero(hist_f, zeros_f)

            def m2(j, c2):
                valid, xv, sv = gather_keys(j, offm)
                e = jnp.exp(xv - mxs)
                plsc.addupdate_scatter(
                    hist_f, [lanebase + ((sv >> 8) & 0xFF)], e, mask=valid)
                return c2

            lax.fori_loop(0, mtrip, m2, 0)
            b2, ab2 = _select(hist_f, ftgt, jnp.float32(0))
            t3 = ftgt - ab2
            _zero(hist_f, zeros_f)

            def m3(j, c2):
                valid, xv, sv = gather_keys(j, offm)
                m = valid & (((sv >> 8) & 0xFF) == b2)
                e = jnp.exp(xv - mxs)
                plsc.addupdate_scatter(
                    hist_f, [lanebase + (sv & 0xFF)], e, mask=m)
                return c2

            lax.fori_loop(0, mtrip, m3, 0)
            b3, _ = _select(hist_f, t3, jnp.float32(0))
            return (b2 << 8) | b3

        def slow_m():
            pref_l, tg = prefm16, ftgt
            for lev in (2, 3):
                sp, sb = 32 - 8 * lev, 24 - 8 * lev
                _zero(hist_f, zeros_f)

                def pm(i, c2, sp=sp, sb=sb, pref_l=pref_l):
                    x = xbuf[pl.ds(i * L, L)]
                    s = _fkey(x)
                    m = ((s >> sp) == pref_l) & (s >= s_thr)
                    e = jnp.exp(x - mxs)
                    plsc.addupdate_scatter(
                        hist_f, [lanebase + ((s >> sb) & 0xFF)], e, mask=m)
                    return c2

                _uloop(pm, 0)
                bq, abq = _select(hist_f, tg, jnp.float32(0))
                pref_l = (pref_l << 8) | bq
                tg = tg - abq
            return pref_l & 0xFFFF

        low16m = lax.cond(mtrip <= CPL, fast_m, slow_m)
        s_tp = (prefm16 << 16) | low16m

        smaxv = jnp.full((L,), smax)

        def po(i, c2):
            x = xbuf[pl.ds(i * L, L)]
            s = _fkey(x)
            keep = (s >= s_tp) | (s == smaxv)
            xbuf[pl.ds(i * L, L)] = jnp.where(keep, x, neg_inf)
            return c2

        _uloop(po, 0)
        pltpu.sync_copy(xbuf, out.at[row])
        return c

    lax.fori_loop(0, RPW, row_body, 0)


_call = functools.partial(
    pl.kernel,
    out_type=jax.ShapeDtypeStruct((B, V), jnp.float32),
    mesh=plsc.VectorSubcoreMesh(core_axis_name="c", subcore_axis_name="s",
                                num_cores=NC, num_subcores=NS),
    scratch_types=[
        pltpu.VMEM((V,), jnp.float32),
        pltpu.VMEM((HSZ,), jnp.int32),
        pltpu.VMEM((HSZ,), jnp.float32),
        pltpu.VMEM((CAP,), jnp.int32),
        pltpu.VMEM((3 * L,), jnp.float32),
    ],
    compiler_params=pltpu.CompilerParams(needs_layout_passes=False),
)(_body)


def kernel(logits, temperatures, k, p):
    t = temperatures.astype(jnp.float32)
    kf = jax.lax.bitcast_convert_type(k.astype(jnp.int32), jnp.float32)
    pf = p.astype(jnp.float32)
    scal = jnp.stack([t, kf, pf], axis=1)
    scal = jnp.broadcast_to(scal[:, :, None], (B, 3, L)).reshape(B, 3 * L)
    return _call(logits.astype(jnp.float32), scal)


# speedup vs baseline: 1.1594x; 1.0846x over previous
import functools

import jax
import jax.numpy as jnp
from jax import lax
from jax.experimental import pallas as pl
from jax.experimental.pallas import tpu as pltpu
from jax.experimental.pallas import tpu_sc as plsc

B = 128
V = 100000
NC, NS, L = 2, 16, 16
NW = NC * NS
RPW = B // NW
NVEC = V // L
NB = 256
HSZ = NB * L
CPL = 768
CAP = (L - 1) * CPL + NVEC + L
UNROLL = 10


def _fkey(x):
    b = plsc.bitcast(x, jnp.int32)
    return jnp.where(b < 0, (b & 0x7FFFFFFF) ^ (-1), b)


def _unkey(s):
    b = jnp.where(s < 0, (s ^ (-1)) | jnp.int32(-(2 ** 31)), s)
    return plsc.bitcast(b, jnp.float32)


def _uloop(body, init):
    if isinstance(init, int):
        init = jnp.int32(init)
    return plsc.parallel_loop(0, NVEC, 1, unroll=UNROLL, carry=init)(body)


def _zero(ref, zval):
    def bd(i, c):
        ref[pl.ds(i * L, L)] = zval
        return c
    plsc.parallel_loop(0, HSZ // L, 1, unroll=8, carry=jnp.int32(0))(bd)


def _select(ref, target, zscal):
    zv = jnp.full((L,), zscal)
    onev = jnp.ones((L,), jnp.int32)
    zci = jnp.zeros((L,), jnp.int32)

    def bd(jj, c):
        carry, cntv, abv = c
        j = 15 - jj
        tot = zv
        for l in range(L):
            tot = tot + ref[pl.ds(l * NB + j * L, L)]
        csum = lax.rev(plsc.cumsum(lax.rev(tot, (0,))), (0,))
        cum = csum + carry
        mask = cum >= target
        cntv = cntv + jnp.where(mask, onev, zci)
        abv = jnp.maximum(abv, jnp.where(mask, zv, cum))
        return (jnp.max(cum), cntv, abv)

    total, cntv, abv = lax.fori_loop(0, 16, bd, (zscal, zci, zv))
    bkt = jnp.maximum(jnp.sum(cntv) - 1, 0)
    return bkt, jnp.max(abv)


def _body(lg, scal, out, xbuf, hist_i, hist_f, cand, ssp):
    cid = lax.axis_index("c")
    sid = lax.axis_index("s")
    wid = sid * NC + cid
    lanes = lax.iota(jnp.int32, L)
    lanebase = lanes * NB
    lanebase128 = lanebase + 128
    lanesC = lanes * CPL
    ones_i = jnp.ones((L,), jnp.int32)
    zeros_i = jnp.zeros((L,), jnp.int32)
    zeros_f = jnp.zeros((L,), jnp.float32)
    neg_inf = jnp.full((L,), -jnp.inf, jnp.float32)

    def row_body(r, c):
        row = wid * RPW + r
        pltpu.sync_copy(lg.at[row], xbuf)
        pltpu.sync_copy(scal.at[row], ssp)
        tval = ssp[pl.ds(0, L)]
        kt = jnp.max(plsc.bitcast(ssp[pl.ds(L, L)], jnp.int32))
        ps = jnp.max(ssp[pl.ds(2 * L, L)])

        _zero(hist_i, zeros_i)

        def p1(i, mx):
            v = xbuf[pl.ds(i * L, L)]
            x = v / tval
            xbuf[pl.ds(i * L, L)] = x
            s = _fkey(x)
            plsc.addupdate_scatter(hist_i, [lanebase128 + (s >> 24)], ones_i)
            return jnp.maximum(mx, x)

        mxv = _uloop(p1, neg_inf)
        mxs = jnp.max(mxv)
        smax = jnp.max(_fkey(jnp.full((L,), mxs)))

        def key_at(i):
            return _fkey(xbuf[pl.ds(i * L, L)])

        bkt, above = _select(hist_i, kt, jnp.int32(0))
        pref = bkt - 128
        tgt = kt - above
        _zero(hist_i, zeros_i)

        def pc1(i, c2, pref=pref):
            s = key_at(i)
            m = (s >> 24) == pref
            plsc.addupdate_scatter(
                hist_i, [lanebase + ((s >> 16) & 0xFF)], ones_i, mask=m)
            return c2

        _uloop(pc1, 0)
        bkt, above = _select(hist_i, tgt, jnp.int32(0))
        pref16 = (pref << 8) | bkt
        tgt = tgt - above

        def ccomp(i, off, pref16=pref16):
            s = key_at(i)
            m = (s >> 16) == pref16
            plsc.store_scatter(cand, [lanesC + off], i * L + lanes, mask=m)
            return off + jnp.where(m, ones_i, zeros_i)

        offv = _uloop(ccomp, zeros_i)
        ntrip = jnp.max(offv)

        def gather_keys(j, offs):
            valid = j < offs
            idxv = plsc.load_gather(cand, [lanesC + j], mask=valid)
            xv = plsc.load_gather(xbuf, [idxv], mask=valid)
            return valid, xv, _fkey(xv)

        def fast_c():
            _zero(hist_i, zeros_i)

            def l2(j, c2):
                valid, _, sv = gather_keys(j, offv)
                plsc.addupdate_scatter(
                    hist_i, [lanebase + ((sv >> 8) & 0xFF)], ones_i,
                    mask=valid)
                return c2

            lax.fori_loop(0, ntrip, l2, 0)
            b2, ab2 = _select(hist_i, tgt, jnp.int32(0))
            t3 = tgt - ab2
            _zero(hist_i, zeros_i)

            def l3(j, c2):
                valid, _, sv = gather_keys(j, offv)
                m = valid & (((sv >> 8) & 0xFF) == b2)
                plsc.addupdate_scatter(
                    hist_i, [lanebase + (sv & 0xFF)], ones_i, mask=m)
                return c2

            lax.fori_loop(0, ntrip, l3, 0)
            b3, _ = _select(hist_i, t3, jnp.int32(0))
            return (b2 << 8) | b3

        def slow_c():
            pref_l, tg = pref16, tgt
            for lev in (2, 3):
                sp, sb = 32 - 8 * lev, 24 - 8 * lev
                _zero(hist_i, zeros_i)

                def pc(i, c2, sp=sp, sb=sb, pref_l=pref_l):
                    s = key_at(i)
                    m = (s >> sp) == pref_l
                    plsc.addupdate_scatter(
                        hist_i, [lanebase + ((s >> sb) & 0xFF)], ones_i,
                        mask=m)
                    return c2

                _uloop(pc, 0)
                bq, abq = _select(hist_i, tg, jnp.int32(0))
                pref_l = (pref_l << 8) | bq
                tg = tg - abq
            return pref_l & 0xFFFF

        low16 = lax.cond(ntrip <= CPL, fast_c, slow_c)
        s_thr = (pref16 << 16) | low16

        _zero(hist_f, zeros_f)

        def pm0(i, zacc):
            x = xbuf[pl.ds(i * L, L)]
            s = _fkey(x)
            m = s >= s_thr
            e = jnp.exp(x - mxs)
            plsc.addupdate_scatter(hist_f, [lanebase128 + (s >> 24)],
                                   e, mask=m)
            return zacc + jnp.where(m, e, zeros_f)

        zv = _uloop(pm0, zeros_f)
        ptarget = ps * jnp.sum(zv)
        bktm, fabove = _select(hist_f, ptarget, jnp.float32(0))
        prefm = bktm - 128
        ftgt = ptarget - fabove
        _zero(hist_f, zeros_f)

        plo = jnp.maximum(s_thr, prefm << 24)
        phi = (prefm << 24) + 0x1000000

        def pm1(i, c2):
            x = xbuf[pl.ds(i * L, L)]
            s = _fkey(x)
            m = (s >= plo) & (s < phi)
            e = jnp.exp(x - mxs)
            plsc.addupdate_scatter(
                hist_f, [lanebase + ((s >> 16) & 0xFF)], e, mask=m)
            return c2

        _uloop(pm1, 0)
        bktm, fabove = _select(hist_f, ftgt, jnp.float32(0))
        prefm16 = (prefm << 8) | bktm
        ftgt = ftgt - fabove

        mlo = jnp.maximum(s_thr, prefm16 << 16)
        mhi = (prefm16 << 16) + 0x10000

        def mcomp(i, off):
            s = key_at(i)
            m = (s >= mlo) & (s < mhi)
            plsc.store_scatter(cand, [lanesC + off], i * L + lanes, mask=m)
            return off + jnp.where(m, ones_i, zeros_i)

        offm = _uloop(mcomp, zeros_i)
        mtrip = jnp.max(offm)

        def fast_m():
            _zero(hist_f, zeros_f)

            def m2(j, c2):
                valid, xv, sv = gather_keys(j, offm)
                e = jnp.exp(xv - mxs)
                plsc.addupdate_scatter(
                    hist_f, [lanebase + ((sv >> 8) & 0xFF)], e, mask=valid)
                return c2

            lax.fori_loop(0, mtrip, m2, 0)
            b2, ab2 = _select(hist_f, ftgt, jnp.float32(0))
            t3 = ftgt - ab2
            _zero(hist_f, zeros_f)

            def m3(j, c2):
                valid, xv, sv = gather_keys(j, offm)
                m = valid & (((sv >> 8) & 0xFF) == b2)
                e = jnp.exp(xv - mxs)
                plsc.addupdate_scatter(
                    hist_f, [lanebase + (sv & 0xFF)], e, mask=m)
                return c2

            lax.fori_loop(0, mtrip, m3, 0)
            b3, _ = _select(hist_f, t3, jnp.float32(0))
            return (b2 << 8) | b3

        def slow_m():
            pref_l, tg = prefm16, ftgt
            for lev in (2, 3):
                sp, sb = 32 - 8 * lev, 24 - 8 * lev
                _zero(hist_f, zeros_f)

                def pm(i, c2, sp=sp, sb=sb, pref_l=pref_l):
                    x = xbuf[pl.ds(i * L, L)]
                    s = _fkey(x)
                    m = ((s >> sp) == pref_l) & (s >= s_thr)
                    e = jnp.exp(x - mxs)
                    plsc.addupdate_scatter(
                        hist_f, [lanebase + ((s >> sb) & 0xFF)], e, mask=m)
                    return c2

                _uloop(pm, 0)
                bq, abq = _select(hist_f, tg, jnp.float32(0))
                pref_l = (pref_l << 8) | bq
                tg = tg - abq
            return pref_l & 0xFFFF

        low16m = lax.cond(mtrip <= CPL, fast_m, slow_m)
        s_tp = (prefm16 << 16) | low16m

        smaxv = jnp.full((L,), smax)

        def po(i, c2):
            x = xbuf[pl.ds(i * L, L)]
            s = _fkey(x)
            keep = (s >= s_tp) | (s == smaxv)
            xbuf[pl.ds(i * L, L)] = jnp.where(keep, x, neg_inf)
            return c2

        _uloop(po, 0)
        pltpu.sync_copy(xbuf, out.at[row])
        return c

    lax.fori_loop(0, RPW, row_body, 0)


_call = functools.partial(
    pl.kernel,
    out_type=jax.ShapeDtypeStruct((B, V), jnp.float32),
    mesh=plsc.VectorSubcoreMesh(core_axis_name="c", subcore_axis_name="s",
                                num_cores=NC, num_subcores=NS),
    scratch_types=[
        pltpu.VMEM((V,), jnp.float32),
        pltpu.VMEM((HSZ,), jnp.int32),
        pltpu.VMEM((HSZ,), jnp.float32),
        pltpu.VMEM((CAP,), jnp.int32),
        pltpu.VMEM((3 * L,), jnp.float32),
    ],
    compiler_params=pltpu.CompilerParams(needs_layout_passes=False),
)(_body)


def kernel(logits, temperatures, k, p):
    t = temperatures.astype(jnp.float32)
    kf = jax.lax.bitcast_convert_type(k.astype(jnp.int32), jnp.float32)
    pf = p.astype(jnp.float32)
    scal = jnp.stack([t, kf, pf], axis=1)
    scal = jnp.broadcast_to(scal[:, :, None], (B, 3, L)).reshape(B, 3 * L)
    return _call(logits.astype(jnp.float32), scal)


# speedup vs baseline: 1.2213x; 1.0534x over previous
import functools

import jax
import jax.numpy as jnp
from jax import lax
from jax.experimental import pallas as pl
from jax.experimental.pallas import tpu as pltpu
from jax.experimental.pallas import tpu_sc as plsc

B = 128
V = 100000
NC, NS, L = 2, 16, 16
NW = NC * NS
RPW = B // NW
NVEC = V // L
NB = 256
HSZ = NB * L
CPL = 768
CAP = (L - 1) * CPL + NVEC + L
UNROLL = 10


def _fkey(x):
    b = plsc.bitcast(x, jnp.int32)
    return jnp.where(b < 0, (b & 0x7FFFFFFF) ^ (-1), b)


def _unkey(s):
    b = jnp.where(s < 0, (s ^ (-1)) | jnp.int32(-(2 ** 31)), s)
    return plsc.bitcast(b, jnp.float32)


def _uloop(body, init):
    if isinstance(init, int):
        init = jnp.int32(init)
    return plsc.parallel_loop(0, NVEC, 1, unroll=UNROLL, carry=init)(body)


def _zero(ref, zval):
    def bd(i, c):
        ref[pl.ds(i * L, L)] = zval
        return c
    plsc.parallel_loop(0, HSZ // L, 1, unroll=8, carry=jnp.int32(0))(bd)


def _select(ref, target, zscal):
    zv = jnp.full((L,), zscal)
    onev = jnp.ones((L,), jnp.int32)
    zci = jnp.zeros((L,), jnp.int32)

    def bd(jj, c):
        carry, cntv, abv = c
        j = 15 - jj
        tot = zv
        for l in range(L):
            tot = tot + ref[pl.ds(l * NB + j * L, L)]
        csum = lax.rev(plsc.cumsum(lax.rev(tot, (0,))), (0,))
        cum = csum + carry
        mask = cum >= target
        cntv = cntv + jnp.where(mask, onev, zci)
        abv = jnp.maximum(abv, jnp.where(mask, zv, cum))
        return (jnp.max(cum), cntv, abv)

    total, cntv, abv = lax.fori_loop(0, 16, bd, (zscal, zci, zv))
    bkt = jnp.maximum(jnp.sum(cntv) - 1, 0)
    return bkt, jnp.max(abv)


def _total(ref):
    def bd(j, acc):
        for l in range(L):
            acc = acc + ref[pl.ds(l * NB + j * L, L)]
        return acc
    return jnp.sum(lax.fori_loop(0, 16, bd, jnp.zeros((L,), jnp.float32)))


def _body(lg, scal, out, xbuf, hist_i, hist_f, cand, ssp):
    cid = lax.axis_index("c")
    sid = lax.axis_index("s")
    wid = sid * NC + cid
    lanes = lax.iota(jnp.int32, L)
    lanebase = lanes * NB
    lanebase128 = lanebase + 128
    lanesC = lanes * CPL
    ones_i = jnp.ones((L,), jnp.int32)
    zeros_i = jnp.zeros((L,), jnp.int32)
    zeros_f = jnp.zeros((L,), jnp.float32)
    neg_inf = jnp.full((L,), -jnp.inf, jnp.float32)

    def row_body(r, c):
        row = wid * RPW + r
        pltpu.sync_copy(lg.at[row], xbuf)
        pltpu.sync_copy(scal.at[row], ssp)
        tval = ssp[pl.ds(0, L)]
        kt = jnp.max(plsc.bitcast(ssp[pl.ds(L, L)], jnp.int32))
        ps = jnp.max(ssp[pl.ds(2 * L, L)])

        _zero(hist_i, zeros_i)

        def p1(i, mx):
            v = xbuf[pl.ds(i * L, L)]
            x = v / tval
            xbuf[pl.ds(i * L, L)] = x
            s = _fkey(x)
            plsc.addupdate_scatter(hist_i, [lanebase128 + (s >> 24)], ones_i)
            return jnp.maximum(mx, x)

        mxv = _uloop(p1, neg_inf)
        mxs = jnp.max(mxv)
        smax = jnp.max(_fkey(jnp.full((L,), mxs)))

        def key_at(i):
            return _fkey(xbuf[pl.ds(i * L, L)])

        bkt, above = _select(hist_i, kt, jnp.int32(0))
        pref = bkt - 128
        tgt = kt - above
        _zero(hist_i, zeros_i)

        pc1base = lanebase - (pref << 8)

        def pc1(i, c2, pref=pref):
            s = key_at(i)
            m = (s >> 24) == pref
            plsc.addupdate_scatter(
                hist_i, [pc1base + (s >> 16)], ones_i, mask=m)
            return c2

        _uloop(pc1, 0)
        bkt, above = _select(hist_i, tgt, jnp.int32(0))
        pref16 = (pref << 8) | bkt
        tgt = tgt - above

        def ccomp(i, off, pref16=pref16):
            x = xbuf[pl.ds(i * L, L)]
            m = (_fkey(x) >> 16) == pref16
            plsc.store_scatter(cand, [lanesC + off], x, mask=m)
            return off + jnp.where(m, ones_i, zeros_i)

        offv = _uloop(ccomp, zeros_i)
        ntrip = jnp.max(offv)

        def gather_keys(j, offs):
            valid = j < offs
            xv = plsc.load_gather(cand, [lanesC + j], mask=valid)
            return valid, xv, _fkey(xv)

        def fast_c():
            _zero(hist_i, zeros_i)

            def l2(j, c2):
                valid, _, sv = gather_keys(j, offv)
                plsc.addupdate_scatter(
                    hist_i, [lanebase + ((sv >> 8) & 0xFF)], ones_i,
                    mask=valid)
                return c2

            lax.fori_loop(0, ntrip, l2, 0)
            b2, ab2 = _select(hist_i, tgt, jnp.int32(0))
            t3 = tgt - ab2
            _zero(hist_i, zeros_i)

            def l3(j, c2):
                valid, _, sv = gather_keys(j, offv)
                m = valid & (((sv >> 8) & 0xFF) == b2)
                plsc.addupdate_scatter(
                    hist_i, [lanebase + (sv & 0xFF)], ones_i, mask=m)
                return c2

            lax.fori_loop(0, ntrip, l3, 0)
            b3, _ = _select(hist_i, t3, jnp.int32(0))
            return (b2 << 8) | b3

        def slow_c():
            pref_l, tg = pref16, tgt
            for lev in (2, 3):
                sp, sb = 32 - 8 * lev, 24 - 8 * lev
                _zero(hist_i, zeros_i)

                def pc(i, c2, sp=sp, sb=sb, pref_l=pref_l):
                    s = key_at(i)
                    m = (s >> sp) == pref_l
                    plsc.addupdate_scatter(
                        hist_i, [lanebase + ((s >> sb) & 0xFF)], ones_i,
                        mask=m)
                    return c2

                _uloop(pc, 0)
                bq, abq = _select(hist_i, tg, jnp.int32(0))
                pref_l = (pref_l << 8) | bq
                tg = tg - abq
            return pref_l & 0xFFFF

        low16 = lax.cond(ntrip <= CPL, fast_c, slow_c)
        s_thr = (pref16 << 16) | low16

        _zero(hist_f, zeros_f)

        def pm0(i, c2):
            x = xbuf[pl.ds(i * L, L)]
            s = _fkey(x)
            m = s >= s_thr
            e = jnp.exp(x - mxs)
            plsc.addupdate_scatter(hist_f, [lanebase128 + (s >> 24)],
                                   e, mask=m)
            return c2

        _uloop(pm0, 0)
        ptarget = ps * _total(hist_f)
        bktm, fabove = _select(hist_f, ptarget, jnp.float32(0))
        prefm = bktm - 128
        ftgt = ptarget - fabove
        _zero(hist_f, zeros_f)

        plo = jnp.maximum(s_thr, prefm << 24)
        phi = (prefm << 24) + 0x1000000

        pm1base = lanebase - (prefm << 8)

        def pm1(i, c2):
            x = xbuf[pl.ds(i * L, L)]
            s = _fkey(x)
            m = (s >= plo) & (s < phi)
            e = jnp.exp(x - mxs)
            plsc.addupdate_scatter(
                hist_f, [pm1base + (s >> 16)], e, mask=m)
            return c2

        _uloop(pm1, 0)
        bktm, fabove = _select(hist_f, ftgt, jnp.float32(0))
        prefm16 = (prefm << 8) | bktm
        ftgt = ftgt - fabove

        mlo = jnp.maximum(s_thr, prefm16 << 16)
        mhi = (prefm16 << 16) + 0x10000

        def mcomp(i, off):
            x = xbuf[pl.ds(i * L, L)]
            s = _fkey(x)
            m = (s >= mlo) & (s < mhi)
            plsc.store_scatter(cand, [lanesC + off], x, mask=m)
            return off + jnp.where(m, ones_i, zeros_i)

        offm = _uloop(mcomp, zeros_i)
        mtrip = jnp.max(offm)

        def fast_m():
            _zero(hist_f, zeros_f)

            def m2(j, c2):
                valid, xv, sv = gather_keys(j, offm)
                e = jnp.exp(xv - mxs)
                plsc.addupdate_scatter(
                    hist_f, [lanebase + ((sv >> 8) & 0xFF)], e, mask=valid)
                return c2

            lax.fori_loop(0, mtrip, m2, 0)
            b2, ab2 = _select(hist_f, ftgt, jnp.float32(0))
            t3 = ftgt - ab2
            _zero(hist_f, zeros_f)

            def m3(j, c2):
                valid, xv, sv = gather_keys(j, offm)
                m = valid & (((sv >> 8) & 0xFF) == b2)
                e = jnp.exp(xv - mxs)
                plsc.addupdate_scatter(
                    hist_f, [lanebase + (sv & 0xFF)], e, mask=m)
                return c2

            lax.fori_loop(0, mtrip, m3, 0)
            b3, _ = _select(hist_f, t3, jnp.float32(0))
            return (b2 << 8) | b3

        def slow_m():
            pref_l, tg = prefm16, ftgt
            for lev in (2, 3):
                sp, sb = 32 - 8 * lev, 24 - 8 * lev
                _zero(hist_f, zeros_f)

                def pm(i, c2, sp=sp, sb=sb, pref_l=pref_l):
                    x = xbuf[pl.ds(i * L, L)]
                    s = _fkey(x)
                    m = ((s >> sp) == pref_l) & (s >= s_thr)
                    e = jnp.exp(x - mxs)
                    plsc.addupdate_scatter(
                        hist_f, [lanebase + ((s >> sb) & 0xFF)], e, mask=m)
                    return c2

                _uloop(pm, 0)
                bq, abq = _select(hist_f, tg, jnp.float32(0))
                pref_l = (pref_l << 8) | bq
                tg = tg - abq
            return pref_l & 0xFFFF

        low16m = lax.cond(mtrip <= CPL, fast_m, slow_m)
        s_tp = (prefm16 << 16) | low16m

        smaxv = jnp.full((L,), smax)

        def po(i, c2):
            x = xbuf[pl.ds(i * L, L)]
            s = _fkey(x)
            keep = (s >= s_tp) | (s == smaxv)
            xbuf[pl.ds(i * L, L)] = jnp.where(keep, x, neg_inf)
            return c2

        _uloop(po, 0)
        pltpu.sync_copy(xbuf, out.at[row])
        return c

    lax.fori_loop(0, RPW, row_body, 0)


_call = functools.partial(
    pl.kernel,
    out_type=jax.ShapeDtypeStruct((B, V), jnp.float32),
    mesh=plsc.VectorSubcoreMesh(core_axis_name="c", subcore_axis_name="s",
                                num_cores=NC, num_subcores=NS),
    scratch_types=[
        pltpu.VMEM((V,), jnp.float32),
        pltpu.VMEM((HSZ,), jnp.int32),
        pltpu.VMEM((HSZ,), jnp.float32),
        pltpu.VMEM((CAP,), jnp.float32),
        pltpu.VMEM((3 * L,), jnp.float32),
    ],
    compiler_params=pltpu.CompilerParams(needs_layout_passes=False),
)(_body)


def kernel(logits, temperatures, k, p):
    t = temperatures.astype(jnp.float32)
    kf = jax.lax.bitcast_convert_type(k.astype(jnp.int32), jnp.float32)
    pf = p.astype(jnp.float32)
    scal = jnp.stack([t, kf, pf], axis=1)
    scal = jnp.broadcast_to(scal[:, :, None], (B, 3, L)).reshape(B, 3 * L)
    return _call(logits.astype(jnp.float32), scal)
